# Initial kernel scaffold; baseline (speedup 1.0000x reference)
import jax
import jax.numpy as jnp
from jax.experimental import pallas as pl


def kernel(z_e, embedding):
    raise NotImplementedError("write your pallas kernel here")


# speedup vs baseline: 1.4099x; 1.4099x over previous
import functools

import jax
import jax.numpy as jnp
from jax.experimental import pallas as pl

EMB_DIM = 64
NUM_EMB = 1024
N_TOKENS = 16 * 32 * 32


def _vq_body(x_ref, emb_ref, esq_ref, idx_ref, oh_ref, zq_ref):
    x = x_ref[...]
    emb = emb_ref[...]
    emb_sqr = esq_ref[...]
    z_sqr = jnp.sum(x * x, axis=1, keepdims=True)
    m = jax.lax.dot_general(
        x, emb, (((1,), (1,)), ((), ())),
        preferred_element_type=jnp.float32)
    dist = (emb_sqr + z_sqr) - 2.0 * m
    dmin = jnp.min(dist, axis=1, keepdims=True)
    iota = jax.lax.broadcasted_iota(jnp.int32, dist.shape, 1)
    idx = jnp.min(jnp.where(dist == dmin, iota, NUM_EMB), axis=1)
    idx = idx.astype(jnp.int32)
    oh = (iota == idx[:, None]).astype(jnp.float32)
    zq = jax.lax.dot_general(
        oh, emb, (((1,), (0,)), ((), ())),
        preferred_element_type=jnp.float32)
    idx_ref[...] = idx
    oh_ref[...] = oh
    zq_ref[...] = zq


@functools.partial(jax.jit, static_argnames=())
def kernel(z_e, embedding):
    z = jnp.transpose(z_e, (0, 2, 3, 1))
    z_flat = z.reshape(-1, EMB_DIM)
    emb_sqr = jnp.sum(embedding ** 2, axis=1).reshape(1, NUM_EMB)
    bn = 256
    grid = (N_TOKENS // bn,)
    idx, oh, zq = pl.pallas_call(
        _vq_body,
        grid=grid,
        in_specs=[
            pl.BlockSpec((bn, EMB_DIM), lambda i: (i, 0)),
            pl.BlockSpec((NUM_EMB, EMB_DIM), lambda i: (0, 0)),
            pl.BlockSpec((1, NUM_EMB), lambda i: (0, 0)),
        ],
        out_specs=[
            pl.BlockSpec((bn,), lambda i: (i,)),
            pl.BlockSpec((bn, NUM_EMB), lambda i: (i, 0)),
            pl.BlockSpec((bn, EMB_DIM), lambda i: (i, 0)),
        ],
        out_shape=[
            jax.ShapeDtypeStruct((N_TOKENS,), jnp.int32),
            jax.ShapeDtypeStruct((N_TOKENS, NUM_EMB), jnp.float32),
            jax.ShapeDtypeStruct((N_TOKENS, EMB_DIM), jnp.float32),
        ],
    )(z_flat, embedding, emb_sqr)
    z_q = zq.reshape(z.shape)
    return (z, z_q, idx, oh)


# speedup vs baseline: 1.9019x; 1.3489x over previous
import functools

import jax
import jax.numpy as jnp
from jax.experimental import pallas as pl

EMB_DIM = 64
NUM_EMB = 1024
N_TOKENS = 16 * 32 * 32
BN = 2048


def _vq_body(x_ref, emb_ref, esq_ref, idx_ref, oh_ref, zq_ref):
    x = x_ref[...]
    emb = emb_ref[...]
    emb_sqr = esq_ref[...]
    z_sqr = jnp.sum(x * x, axis=1, keepdims=True)
    m2 = jax.lax.dot_general(
        x + x, emb, (((1,), (1,)), ((), ())),
        preferred_element_type=jnp.float32)
    dist = (emb_sqr + z_sqr) - m2
    dmin = jnp.min(dist, axis=1, keepdims=True)
    iota = jax.lax.broadcasted_iota(jnp.int32, dist.shape, 1)
    idx = jnp.min(jnp.where(dist == dmin, iota, NUM_EMB),
                  axis=1).astype(jnp.int32)
    hit = iota == idx[:, None]
    oh = hit.astype(jnp.float32)
    zq = jax.lax.dot_general(
        oh, emb, (((1,), (0,)), ((), ())),
        preferred_element_type=jnp.float32)
    idx_ref[...] = idx
    oh_ref[...] = oh
    zq_ref[...] = zq


@functools.partial(jax.jit, static_argnames=())
def kernel(z_e, embedding):
    z = jnp.transpose(z_e, (0, 2, 3, 1))
    z_flat = z.reshape(-1, EMB_DIM)
    emb_sqr = jnp.sum(embedding ** 2, axis=1).reshape(1, NUM_EMB)
    grid = (N_TOKENS // BN,)
    idx, oh, zq = pl.pallas_call(
        _vq_body,
        grid=grid,
        in_specs=[
            pl.BlockSpec((BN, EMB_DIM), lambda i: (i, 0)),
            pl.BlockSpec((NUM_EMB, EMB_DIM), lambda i: (0, 0)),
            pl.BlockSpec((1, NUM_EMB), lambda i: (0, 0)),
        ],
        out_specs=[
            pl.BlockSpec((BN,), lambda i: (i,)),
            pl.BlockSpec((BN, NUM_EMB), lambda i: (i, 0)),
            pl.BlockSpec((BN, EMB_DIM), lambda i: (i, 0)),
        ],
        out_shape=[
            jax.ShapeDtypeStruct((N_TOKENS,), jnp.int32),
            jax.ShapeDtypeStruct((N_TOKENS, NUM_EMB), jnp.float32),
            jax.ShapeDtypeStruct((N_TOKENS, EMB_DIM), jnp.float32),
        ],
    )(z_flat, embedding, emb_sqr)
    z_q = zq.reshape(z.shape)
    return (z, z_q, idx, oh)


# speedup vs baseline: 2.0420x; 1.0737x over previous
import functools

import jax
import jax.numpy as jnp
from jax.experimental import pallas as pl

EMB_DIM = 64
NUM_EMB = 1024
N_TOKENS = 16 * 32 * 32
BN = 2048


def _vq_body(x_ref, emb_ref, esq_ref, idx_ref, oh_ref, zq_ref):
    x = x_ref[...]
    emb = emb_ref[...]
    emb_sqr = esq_ref[...]
    z_sqr = jnp.sum(x * x, axis=1, keepdims=True)
    m2 = jax.lax.dot_general(
        x + x, emb, (((1,), (1,)), ((), ())),
        preferred_element_type=jnp.float32)
    dist = (emb_sqr + z_sqr) - m2
    dmin = jnp.min(dist, axis=1, keepdims=True)
    iotaf = jax.lax.broadcasted_iota(jnp.int32, dist.shape, 1
                                     ).astype(jnp.float32)
    idxf = jnp.min(jnp.where(dist == dmin, iotaf, float(NUM_EMB)),
                   axis=1, keepdims=True)
    idx = idxf[:, 0].astype(jnp.int32)
    oh = (iotaf == idxf).astype(jnp.float32)
    zq = jax.lax.dot_general(
        oh, emb, (((1,), (0,)), ((), ())),
        preferred_element_type=jnp.float32)
    idx_ref[...] = idx
    oh_ref[...] = oh
    zq_ref[...] = zq


@functools.partial(jax.jit, static_argnames=())
def kernel(z_e, embedding):
    z = jnp.transpose(z_e, (0, 2, 3, 1))
    z_flat = z.reshape(-1, EMB_DIM)
    emb_sqr = jnp.sum(embedding ** 2, axis=1).reshape(1, NUM_EMB)
    grid = (N_TOKENS // BN,)
    idx, oh, zq = pl.pallas_call(
        _vq_body,
        grid=grid,
        in_specs=[
            pl.BlockSpec((BN, EMB_DIM), lambda i: (i, 0)),
            pl.BlockSpec((NUM_EMB, EMB_DIM), lambda i: (0, 0)),
            pl.BlockSpec((1, NUM_EMB), lambda i: (0, 0)),
        ],
        out_specs=[
            pl.BlockSpec((BN,), lambda i: (i,)),
            pl.BlockSpec((BN, NUM_EMB), lambda i: (i, 0)),
            pl.BlockSpec((BN, EMB_DIM), lambda i: (i, 0)),
        ],
        out_shape=[
            jax.ShapeDtypeStruct((N_TOKENS,), jnp.int32),
            jax.ShapeDtypeStruct((N_TOKENS, NUM_EMB), jnp.float32),
            jax.ShapeDtypeStruct((N_TOKENS, EMB_DIM), jnp.float32),
        ],
    )(z_flat, embedding, emb_sqr)
    z_q = zq.reshape(z.shape)
    return (z, z_q, idx, oh)
